# baseline (speedup 1.0000x reference)
import functools

import jax
import jax.numpy as jnp
from jax import lax
from jax.experimental import pallas as pl
from jax.experimental.pallas import tpu as pltpu
from jax.experimental.pallas import tpu_sc as plsc

N_NODES = 10000
IN_FEAT = 128
HIDDEN = 32
NUM_GRAPHS = 64

NC = 2
NS = 16
NW = NC * NS
CH = 128
CPW = 80
CHUNKS = NW * CPW
PAD_E = CHUNKS * CH
NPAD = N_NODES + 1136
RPS = NPAD // NS


def _kan_pack(W):
    return jnp.concatenate(
        [W[0, :, :, 0].T, W[0, :, :, 1].T, W[1, :, :, 0].T, W[1, :, :, 1].T],
        axis=0)


def _kan_feats(x):
    x2 = 2.0 * x
    return jnp.concatenate(
        [jnp.cos(x), jnp.cos(x2), jnp.sin(x), jnp.sin(x2)], axis=1)


def _stage_a_body(x_ref, wk_ref, wl1_ref, g1_ref):
    h0 = jnp.dot(_kan_feats(x_ref[:]), wk_ref[:],
                 preferred_element_type=jnp.float32)
    g1_ref[:] = jnp.dot(_kan_feats(h0), wl1_ref[:],
                        preferred_element_type=jnp.float32)


def _stage_b_body(p_ref, w_ref, g_ref):
    h = p_ref[0] + p_ref[1]
    g_ref[:] = jnp.dot(_kan_feats(h), w_ref[:],
                       preferred_element_type=jnp.float32)


def _stage_c_body(p_ref, b_ref, w1_ref, w2_ref, out_ref):
    h = p_ref[0] + p_ref[1]
    seg = lax.broadcasted_iota(jnp.int32, (NPAD, NUM_GRAPHS), 1)
    m = (b_ref[:] == seg).astype(jnp.float32)
    y = lax.dot_general(m, h, (((0,), (0,)), ((), ())),
                        preferred_element_type=jnp.float32,
                        precision=lax.Precision.HIGHEST)
    z = jnp.dot(_kan_feats(y), w1_ref[:], preferred_element_type=jnp.float32)
    z = jnp.where(z >= 0, z, 0.01 * z)
    z = jnp.dot(_kan_feats(z), w2_ref[:], preferred_element_type=jnp.float32)
    out_ref[:] = jax.nn.sigmoid(z)


NB = 4
NG = CPW // NB


def _sc_mp_body(g_hbm, src_hbm, dst_hbm, zero_hbm, out_hbm,
                srcs_v, dsts_v, rows_v, stage_v, acc_sh, gsem, ssem):
    c = lax.axis_index("c")
    s = lax.axis_index("s")
    wid = c * NS + s
    pltpu.sync_copy(zero_hbm, stage_v)
    pltpu.sync_copy(stage_v, acc_sh.at[pl.ds(s * RPS, RPS)])
    pltpu.sync_copy(src_hbm.at[pl.ds(wid * CPW, CPW)], srcs_v)
    pltpu.sync_copy(dst_hbm.at[pl.ds(wid * CPW, CPW)], dsts_v)
    plsc.subcore_barrier()

    def fire_gathers(g, half):
        for b in range(NB):
            pltpu.async_copy(g_hbm.at[srcs_v.at[g * NB + b]],
                             rows_v.at[half].at[b], gsem)

    def drain_gathers(g, half):
        for b in range(NB):
            pltpu.make_async_copy(g_hbm.at[srcs_v.at[g * NB + b]],
                                  rows_v.at[half].at[b], gsem).wait()

    def fire_scatters(g, half):
        for b in range(NB):
            pltpu.async_copy(rows_v.at[half].at[b],
                             acc_sh.at[dsts_v.at[g * NB + b]], ssem, add=True)

    def drain_scatters(g, half):
        for b in range(NB):
            pltpu.make_async_copy(rows_v.at[half].at[b],
                                  acc_sh.at[dsts_v.at[g * NB + b]], ssem).wait()

    fire_gathers(0, 0)

    def body(k, carry):
        g0 = 2 * k

        @pl.when(k > 0)
        def _():
            drain_scatters(g0 - 1, 1)

        fire_gathers(g0 + 1, 1)
        drain_gathers(g0, 0)
        fire_scatters(g0, 0)
        drain_scatters(g0, 0)

        @pl.when(k < NG // 2 - 1)
        def _():
            fire_gathers(g0 + 2, 0)

        drain_gathers(g0 + 1, 1)
        fire_scatters(g0 + 1, 1)
        return carry

    lax.fori_loop(0, NG // 2, body, 0)
    drain_scatters(NG - 1, 1)
    plsc.subcore_barrier()
    pltpu.sync_copy(acc_sh.at[pl.ds(s * RPS, RPS)], stage_v)
    pltpu.sync_copy(stage_v, out_hbm.at[c].at[pl.ds(s * RPS, RPS)])


_sc_mp = functools.partial(
    pl.kernel,
    out_type=jax.ShapeDtypeStruct((NC, NPAD, HIDDEN), jnp.float32),
    mesh=plsc.VectorSubcoreMesh(core_axis_name="c", subcore_axis_name="s"),
    compiler_params=pltpu.CompilerParams(use_tc_tiling_on_sc=False),
    scratch_types=[
        pltpu.VMEM((CPW, CH), jnp.int32),
        pltpu.VMEM((CPW, CH), jnp.int32),
        pltpu.VMEM((2, NB, CH, HIDDEN), jnp.float32),
        pltpu.VMEM((RPS, HIDDEN), jnp.float32),
        pltpu.VMEM_SHARED((NPAD, HIDDEN), jnp.float32),
        pltpu.SemaphoreType.DMA,
        pltpu.SemaphoreType.DMA,
    ],
)(_sc_mp_body)


def kernel(x, edge_index, batch, W_kan, W_l1, W_l2, W1, W2):
    wk = _kan_pack(W_kan)
    wl1 = _kan_pack(W_l1)
    wl2 = _kan_pack(W_l2)
    w1 = _kan_pack(W1)
    w2 = _kan_pack(W2)

    src = edge_index[0].astype(jnp.int32)
    dst = edge_index[1].astype(jnp.int32)
    n_e = src.shape[0]
    pad = PAD_E - n_e
    src_p = jnp.concatenate(
        [src, jnp.zeros((pad,), jnp.int32)]).reshape(CHUNKS, CH)
    dst_p = jnp.concatenate(
        [dst, N_NODES + (jnp.arange(pad, dtype=jnp.int32) % (NPAD - N_NODES))]
    ).reshape(CHUNKS, CH)
    zeros = jnp.zeros((RPS, HIDDEN), jnp.float32)
    batch_p = jnp.concatenate(
        [batch.astype(jnp.int32),
         jnp.full((NPAD - N_NODES,), NUM_GRAPHS, jnp.int32)]).reshape(NPAD, 1)

    blk = 2000
    g1 = pl.pallas_call(
        _stage_a_body,
        grid=(N_NODES // blk,),
        in_specs=[
            pl.BlockSpec((blk, IN_FEAT), lambda i: (i, 0)),
            pl.BlockSpec((4 * IN_FEAT, HIDDEN), lambda i: (0, 0)),
            pl.BlockSpec((4 * HIDDEN, HIDDEN), lambda i: (0, 0)),
        ],
        out_specs=pl.BlockSpec((blk, HIDDEN), lambda i: (i, 0)),
        out_shape=jax.ShapeDtypeStruct((N_NODES, HIDDEN), jnp.float32),
    )(x, wk, wl1)

    p1 = _sc_mp(g1, src_p, dst_p, zeros)

    g2 = pl.pallas_call(
        _stage_b_body,
        in_specs=[
            pl.BlockSpec((NC, NPAD, HIDDEN), lambda: (0, 0, 0)),
            pl.BlockSpec((4 * HIDDEN, HIDDEN), lambda: (0, 0)),
        ],
        out_specs=pl.BlockSpec((NPAD, HIDDEN), lambda: (0, 0)),
        out_shape=jax.ShapeDtypeStruct((NPAD, HIDDEN), jnp.float32),
    )(p1, wl2)

    p2 = _sc_mp(g2, src_p, dst_p, zeros)

    out = pl.pallas_call(
        _stage_c_body,
        in_specs=[
            pl.BlockSpec((NC, NPAD, HIDDEN), lambda: (0, 0, 0)),
            pl.BlockSpec((NPAD, 1), lambda: (0, 0)),
            pl.BlockSpec((4 * HIDDEN, HIDDEN), lambda: (0, 0)),
            pl.BlockSpec((4 * HIDDEN, 1), lambda: (0, 0)),
        ],
        out_specs=pl.BlockSpec((NUM_GRAPHS, 1), lambda: (0, 0)),
        out_shape=jax.ShapeDtypeStruct((NUM_GRAPHS, 1), jnp.float32),
    )(p2, batch_p, w1, w2)
    return out


# speedup vs baseline: 1.6793x; 1.6793x over previous
import functools

import jax
import jax.numpy as jnp
from jax import lax
from jax.experimental import pallas as pl
from jax.experimental.pallas import tpu as pltpu
from jax.experimental.pallas import tpu_sc as plsc

N_NODES = 10000
IN_FEAT = 128
HIDDEN = 32
NUM_GRAPHS = 64

NC = 2
NS = 16
NW = NC * NS
CH = 128
CPW = 80
CHUNKS = NW * CPW
PAD_E = CHUNKS * CH
NPAD = N_NODES + 112
RPS = NPAD // NS


def _kan_pack(W):
    return jnp.concatenate(
        [W[0, :, :, 0].T, W[0, :, :, 1].T, W[1, :, :, 0].T, W[1, :, :, 1].T],
        axis=0)


def _kan_feats(x):
    x2 = 2.0 * x
    return jnp.concatenate(
        [jnp.cos(x), jnp.cos(x2), jnp.sin(x), jnp.sin(x2)], axis=1)


def _stage_a_body(x_ref, wk_ref, wl1_ref, g1_ref):
    h0 = jnp.dot(_kan_feats(x_ref[:]), wk_ref[:],
                 preferred_element_type=jnp.float32)
    g1_ref[:] = jnp.dot(_kan_feats(h0), wl1_ref[:],
                        preferred_element_type=jnp.float32)


def _stage_b_body(p_ref, w_ref, g_ref):
    h = p_ref[0] + p_ref[1]
    g_ref[:] = jnp.dot(_kan_feats(h), w_ref[:],
                       preferred_element_type=jnp.float32)


def _stage_c_body(p_ref, b_ref, w1_ref, w2_ref, out_ref):
    h = p_ref[0] + p_ref[1]
    seg = lax.broadcasted_iota(jnp.int32, (NPAD, NUM_GRAPHS), 1)
    m = (b_ref[:] == seg).astype(jnp.float32)
    y = lax.dot_general(m, h, (((0,), (0,)), ((), ())),
                        preferred_element_type=jnp.float32,
                        precision=lax.Precision.HIGHEST)
    z = jnp.dot(_kan_feats(y), w1_ref[:], preferred_element_type=jnp.float32)
    z = jnp.where(z >= 0, z, 0.01 * z)
    z = jnp.dot(_kan_feats(z), w2_ref[:], preferred_element_type=jnp.float32)
    out_ref[:] = jax.nn.sigmoid(z)


NB = 4
NG = CPW // NB


def _sc_mp_body(g_hbm, src_hbm, dst_hbm, zero_hbm, out_hbm,
                srcs_v, dsts_v, rows_v, stage_v, acc_sh, gsem, ssem):
    c = lax.axis_index("c")
    s = lax.axis_index("s")
    wid = c * NS + s
    pltpu.sync_copy(zero_hbm, stage_v)
    pltpu.sync_copy(stage_v, acc_sh.at[pl.ds(s * RPS, RPS)])
    pltpu.sync_copy(src_hbm.at[pl.ds(wid * CPW, CPW)], srcs_v)
    pltpu.sync_copy(dst_hbm.at[pl.ds(wid * CPW, CPW)], dsts_v)
    plsc.subcore_barrier()

    def fire_gathers(g, half):
        for b in range(NB):
            pltpu.async_copy(g_hbm.at[srcs_v.at[g * NB + b]],
                             rows_v.at[half].at[b], gsem)

    def drain_gathers(g, half):
        for b in range(NB):
            pltpu.make_async_copy(g_hbm.at[srcs_v.at[g * NB + b]],
                                  rows_v.at[half].at[b], gsem).wait()

    def fire_scatters(g, half):
        for b in range(NB):
            pltpu.async_copy(rows_v.at[half].at[b],
                             acc_sh.at[dsts_v.at[g * NB + b]], ssem, add=True)

    def drain_scatters(g, half):
        for b in range(NB):
            pltpu.make_async_copy(rows_v.at[half].at[b],
                                  acc_sh.at[dsts_v.at[g * NB + b]], ssem).wait()

    fire_gathers(0, 0)

    def body(k, carry):
        g0 = 2 * k

        @pl.when(k > 0)
        def _():
            drain_scatters(g0 - 1, 1)

        fire_gathers(g0 + 1, 1)
        drain_gathers(g0, 0)
        fire_scatters(g0, 0)
        drain_scatters(g0, 0)

        @pl.when(k < NG // 2 - 1)
        def _():
            fire_gathers(g0 + 2, 0)

        drain_gathers(g0 + 1, 1)
        fire_scatters(g0 + 1, 1)
        return carry

    lax.fori_loop(0, NG // 2, body, 0)
    drain_scatters(NG - 1, 1)
    plsc.subcore_barrier()
    pltpu.sync_copy(acc_sh.at[pl.ds(s * RPS, RPS)], stage_v)
    pltpu.sync_copy(stage_v, out_hbm.at[c].at[pl.ds(s * RPS, RPS)])


_sc_mp = functools.partial(
    pl.kernel,
    out_type=jax.ShapeDtypeStruct((NC, NPAD, HIDDEN), jnp.float32),
    mesh=plsc.VectorSubcoreMesh(core_axis_name="c", subcore_axis_name="s"),
    compiler_params=pltpu.CompilerParams(use_tc_tiling_on_sc=False),
    scratch_types=[
        pltpu.VMEM((CPW, CH), jnp.int32),
        pltpu.VMEM((CPW, CH), jnp.int32),
        pltpu.VMEM((2, NB, CH, HIDDEN), jnp.float32),
        pltpu.VMEM((RPS, HIDDEN), jnp.float32),
        pltpu.VMEM_SHARED((NPAD, HIDDEN), jnp.float32),
        pltpu.SemaphoreType.DMA,
        pltpu.SemaphoreType.DMA,
    ],
)(_sc_mp_body)


def kernel(x, edge_index, batch, W_kan, W_l1, W_l2, W1, W2):
    wk = _kan_pack(W_kan)
    wl1 = _kan_pack(W_l1)
    wl2 = _kan_pack(W_l2)
    w1 = _kan_pack(W1)
    w2 = _kan_pack(W2)

    src = edge_index[0].astype(jnp.int32)
    dst = edge_index[1].astype(jnp.int32)
    n_e = src.shape[0]
    pad = PAD_E - n_e
    def _chunked(a):
        return a.reshape(CPW, NW, CH).transpose(1, 0, 2).reshape(CHUNKS, CH)
    src_p = _chunked(jnp.concatenate(
        [src, jnp.arange(pad, dtype=jnp.int32) % N_NODES]))
    dst_p = _chunked(jnp.concatenate(
        [dst, N_NODES + (jnp.arange(pad, dtype=jnp.int32) % (NPAD - N_NODES))]))
    zeros = jnp.zeros((RPS, HIDDEN), jnp.float32)
    batch_p = jnp.concatenate(
        [batch.astype(jnp.int32),
         jnp.full((NPAD - N_NODES,), NUM_GRAPHS, jnp.int32)]).reshape(NPAD, 1)

    blk = 2000
    g1 = pl.pallas_call(
        _stage_a_body,
        grid=(N_NODES // blk,),
        in_specs=[
            pl.BlockSpec((blk, IN_FEAT), lambda i: (i, 0)),
            pl.BlockSpec((4 * IN_FEAT, HIDDEN), lambda i: (0, 0)),
            pl.BlockSpec((4 * HIDDEN, HIDDEN), lambda i: (0, 0)),
        ],
        out_specs=pl.BlockSpec((blk, HIDDEN), lambda i: (i, 0)),
        out_shape=jax.ShapeDtypeStruct((N_NODES, HIDDEN), jnp.float32),
    )(x, wk, wl1)

    p1 = _sc_mp(g1, src_p, dst_p, zeros)

    g2 = pl.pallas_call(
        _stage_b_body,
        in_specs=[
            pl.BlockSpec((NC, NPAD, HIDDEN), lambda: (0, 0, 0)),
            pl.BlockSpec((4 * HIDDEN, HIDDEN), lambda: (0, 0)),
        ],
        out_specs=pl.BlockSpec((NPAD, HIDDEN), lambda: (0, 0)),
        out_shape=jax.ShapeDtypeStruct((NPAD, HIDDEN), jnp.float32),
    )(p1, wl2)

    p2 = _sc_mp(g2, src_p, dst_p, zeros)

    out = pl.pallas_call(
        _stage_c_body,
        in_specs=[
            pl.BlockSpec((NC, NPAD, HIDDEN), lambda: (0, 0, 0)),
            pl.BlockSpec((NPAD, 1), lambda: (0, 0)),
            pl.BlockSpec((4 * HIDDEN, HIDDEN), lambda: (0, 0)),
            pl.BlockSpec((4 * HIDDEN, 1), lambda: (0, 0)),
        ],
        out_specs=pl.BlockSpec((NUM_GRAPHS, 1), lambda: (0, 0)),
        out_shape=jax.ShapeDtypeStruct((NUM_GRAPHS, 1), jnp.float32),
    )(p2, batch_p, w1, w2)
    return out


# speedup vs baseline: 1.6946x; 1.0091x over previous
import functools

import jax
import jax.numpy as jnp
from jax import lax
from jax.experimental import pallas as pl
from jax.experimental.pallas import tpu as pltpu
from jax.experimental.pallas import tpu_sc as plsc

N_NODES = 10000
IN_FEAT = 128
HIDDEN = 32
NUM_GRAPHS = 64

NC = 2
NS = 16
NW = NC * NS
CH = 128
CPW = 80
CHUNKS = NW * CPW
PAD_E = CHUNKS * CH
NPAD = N_NODES + 112
RPS = NPAD // NS


def _kan_pack(W):
    return jnp.concatenate(
        [W[0, :, :, 0].T, W[0, :, :, 1].T, W[1, :, :, 0].T, W[1, :, :, 1].T],
        axis=0)


def _kan_feats(x):
    x2 = 2.0 * x
    return jnp.concatenate(
        [jnp.cos(x), jnp.cos(x2), jnp.sin(x), jnp.sin(x2)], axis=1)


def _stage_a_body(x_ref, wk_ref, wl1_ref, g1_ref):
    h0 = jnp.dot(_kan_feats(x_ref[:]), wk_ref[:],
                 preferred_element_type=jnp.float32)
    g1_ref[:] = jnp.dot(_kan_feats(h0), wl1_ref[:],
                        preferred_element_type=jnp.float32)


def _stage_b_body(p_ref, w_ref, g_ref):
    h = p_ref[0] + p_ref[1]
    g_ref[:] = jnp.dot(_kan_feats(h), w_ref[:],
                       preferred_element_type=jnp.float32)


def _stage_c_body(p_ref, b_ref, w1_ref, w2_ref, out_ref):
    h = p_ref[0] + p_ref[1]
    seg = lax.broadcasted_iota(jnp.int32, (NPAD, NUM_GRAPHS), 1)
    m = (b_ref[:] == seg).astype(jnp.float32)
    y = lax.dot_general(m, h, (((0,), (0,)), ((), ())),
                        preferred_element_type=jnp.float32,
                        precision=lax.Precision.HIGHEST)
    z = jnp.dot(_kan_feats(y), w1_ref[:], preferred_element_type=jnp.float32)
    z = jnp.where(z >= 0, z, 0.01 * z)
    z = jnp.dot(_kan_feats(z), w2_ref[:], preferred_element_type=jnp.float32)
    out_ref[:] = jax.nn.sigmoid(z)


NB = 4
NG = CPW // NB


def _sc_mp_body(g_hbm, src_hbm, dst_hbm, zero_hbm, out_hbm,
                srcs_v, dsts_v, rows_v, stage_v, acc_sh, gsem, ssem):
    c = lax.axis_index("c")
    s = lax.axis_index("s")
    wid = c * NS + s
    pltpu.sync_copy(zero_hbm, stage_v)
    pltpu.sync_copy(stage_v, acc_sh.at[pl.ds(s * RPS, RPS)])
    pltpu.sync_copy(src_hbm.at[:, wid], srcs_v)
    pltpu.sync_copy(dst_hbm.at[:, wid], dsts_v)
    plsc.subcore_barrier()

    def fire_gathers(g, half):
        for b in range(NB):
            pltpu.async_copy(g_hbm.at[srcs_v.at[g * NB + b]],
                             rows_v.at[half].at[b], gsem)

    def drain_gathers(g, half):
        for b in range(NB):
            pltpu.make_async_copy(g_hbm.at[srcs_v.at[g * NB + b]],
                                  rows_v.at[half].at[b], gsem).wait()

    def fire_scatters(g, half):
        for b in range(NB):
            pltpu.async_copy(rows_v.at[half].at[b],
                             acc_sh.at[dsts_v.at[g * NB + b]], ssem, add=True)

    def drain_scatters(g, half):
        for b in range(NB):
            pltpu.make_async_copy(rows_v.at[half].at[b],
                                  acc_sh.at[dsts_v.at[g * NB + b]], ssem).wait()

    fire_gathers(0, 0)

    def body(k, carry):
        g0 = 2 * k

        @pl.when(k > 0)
        def _():
            drain_scatters(g0 - 1, 1)

        fire_gathers(g0 + 1, 1)
        drain_gathers(g0, 0)
        fire_scatters(g0, 0)
        drain_scatters(g0, 0)

        @pl.when(k < NG // 2 - 1)
        def _():
            fire_gathers(g0 + 2, 0)

        drain_gathers(g0 + 1, 1)
        fire_scatters(g0 + 1, 1)
        return carry

    lax.fori_loop(0, NG // 2, body, 0)
    drain_scatters(NG - 1, 1)
    plsc.subcore_barrier()
    pltpu.sync_copy(acc_sh.at[pl.ds(s * RPS, RPS)], stage_v)
    pltpu.sync_copy(stage_v, out_hbm.at[c].at[pl.ds(s * RPS, RPS)])


_sc_mp = functools.partial(
    pl.kernel,
    out_type=jax.ShapeDtypeStruct((NC, NPAD, HIDDEN), jnp.float32),
    mesh=plsc.VectorSubcoreMesh(core_axis_name="c", subcore_axis_name="s"),
    compiler_params=pltpu.CompilerParams(use_tc_tiling_on_sc=False),
    scratch_types=[
        pltpu.VMEM((CPW, CH), jnp.int32),
        pltpu.VMEM((CPW, CH), jnp.int32),
        pltpu.VMEM((2, NB, CH, HIDDEN), jnp.float32),
        pltpu.VMEM((RPS, HIDDEN), jnp.float32),
        pltpu.VMEM_SHARED((NPAD, HIDDEN), jnp.float32),
        pltpu.SemaphoreType.DMA,
        pltpu.SemaphoreType.DMA,
    ],
)(_sc_mp_body)


def kernel(x, edge_index, batch, W_kan, W_l1, W_l2, W1, W2):
    wk = _kan_pack(W_kan)
    wl1 = _kan_pack(W_l1)
    wl2 = _kan_pack(W_l2)
    w1 = _kan_pack(W1)
    w2 = _kan_pack(W2)

    src = edge_index[0].astype(jnp.int32)
    dst = edge_index[1].astype(jnp.int32)
    n_e = src.shape[0]
    pad = PAD_E - n_e
    src_p = jnp.concatenate(
        [src, jnp.arange(pad, dtype=jnp.int32) % N_NODES]).reshape(CPW, NW, CH)
    dst_p = jnp.concatenate(
        [dst, N_NODES + (jnp.arange(pad, dtype=jnp.int32) % (NPAD - N_NODES))]
    ).reshape(CPW, NW, CH)
    zeros = jnp.zeros((RPS, HIDDEN), jnp.float32)
    batch_p = jnp.concatenate(
        [batch.astype(jnp.int32),
         jnp.full((NPAD - N_NODES,), NUM_GRAPHS, jnp.int32)]).reshape(NPAD, 1)

    blk = 2000
    g1 = pl.pallas_call(
        _stage_a_body,
        grid=(N_NODES // blk,),
        in_specs=[
            pl.BlockSpec((blk, IN_FEAT), lambda i: (i, 0)),
            pl.BlockSpec((4 * IN_FEAT, HIDDEN), lambda i: (0, 0)),
            pl.BlockSpec((4 * HIDDEN, HIDDEN), lambda i: (0, 0)),
        ],
        out_specs=pl.BlockSpec((blk, HIDDEN), lambda i: (i, 0)),
        out_shape=jax.ShapeDtypeStruct((N_NODES, HIDDEN), jnp.float32),
    )(x, wk, wl1)

    p1 = _sc_mp(g1, src_p, dst_p, zeros)

    g2 = pl.pallas_call(
        _stage_b_body,
        in_specs=[
            pl.BlockSpec((NC, NPAD, HIDDEN), lambda: (0, 0, 0)),
            pl.BlockSpec((4 * HIDDEN, HIDDEN), lambda: (0, 0)),
        ],
        out_specs=pl.BlockSpec((NPAD, HIDDEN), lambda: (0, 0)),
        out_shape=jax.ShapeDtypeStruct((NPAD, HIDDEN), jnp.float32),
    )(p1, wl2)

    p2 = _sc_mp(g2, src_p, dst_p, zeros)

    out = pl.pallas_call(
        _stage_c_body,
        in_specs=[
            pl.BlockSpec((NC, NPAD, HIDDEN), lambda: (0, 0, 0)),
            pl.BlockSpec((NPAD, 1), lambda: (0, 0)),
            pl.BlockSpec((4 * HIDDEN, HIDDEN), lambda: (0, 0)),
            pl.BlockSpec((4 * HIDDEN, 1), lambda: (0, 0)),
        ],
        out_specs=pl.BlockSpec((NUM_GRAPHS, 1), lambda: (0, 0)),
        out_shape=jax.ShapeDtypeStruct((NUM_GRAPHS, 1), jnp.float32),
    )(p2, batch_p, w1, w2)
    return out


# speedup vs baseline: 2.3840x; 1.4068x over previous
import functools

import jax
import jax.numpy as jnp
from jax import lax
from jax.experimental import pallas as pl
from jax.experimental.pallas import tpu as pltpu
from jax.experimental.pallas import tpu_sc as plsc

N_NODES = 10000
IN_FEAT = 128
HIDDEN = 32
NUM_GRAPHS = 64

NC = 2
NS = 16
NW = NC * NS
CH = 128
CPW = 80
CHUNKS = NW * CPW
PAD_E = CHUNKS * CH
NPAD = N_NODES + 240
RPS = NPAD // NS


def _kan_pack(W):
    wc = jnp.concatenate([W[0, :, :, 0].T, W[0, :, :, 1].T], axis=0)
    ws = jnp.concatenate([W[1, :, :, 0].T, W[1, :, :, 1].T], axis=0)
    return wc, ws


def _kan_pack_blockdiag(W):
    eye4 = jnp.eye(4, dtype=jnp.float32)
    return tuple(jnp.kron(eye4, W[h, :, :, g].T)
                 for h in (0, 1) for g in (0, 1))


def _kan_feats(x):
    x2 = 2.0 * x
    return (jnp.concatenate([jnp.cos(x), jnp.cos(x2)], axis=1),
            jnp.concatenate([jnp.sin(x), jnp.sin(x2)], axis=1))


def _kan_packed(hp, wb):
    hp2 = 2.0 * hp
    c = jnp.dot(jnp.cos(hp), wb[0], preferred_element_type=jnp.float32)
    c = c + jnp.dot(jnp.cos(hp2), wb[1], preferred_element_type=jnp.float32)
    s = jnp.dot(jnp.sin(hp), wb[2], preferred_element_type=jnp.float32)
    s = s + jnp.dot(jnp.sin(hp2), wb[3], preferred_element_type=jnp.float32)
    return c + s


def _stage_a_body(x_ref, wkc_ref, wks_ref, wl1t_ref, g1_ref):
    fc, fs = _kan_feats(x_ref[:])
    h0t = (lax.dot_general(wkc_ref[:], fc, (((0,), (1,)), ((), ())),
                           preferred_element_type=jnp.float32)
           + lax.dot_general(wks_ref[:], fs, (((0,), (1,)), ((), ())),
                             preferred_element_type=jnp.float32))
    h0t2 = 2.0 * h0t
    g1t = jnp.dot(wl1t_ref[0], jnp.cos(h0t), preferred_element_type=jnp.float32)
    g1t = g1t + jnp.dot(wl1t_ref[1], jnp.cos(h0t2),
                        preferred_element_type=jnp.float32)
    g1t = g1t + jnp.dot(wl1t_ref[2], jnp.sin(h0t),
                        preferred_element_type=jnp.float32)
    g1t = g1t + jnp.dot(wl1t_ref[3], jnp.sin(h0t2),
                        preferred_element_type=jnp.float32)
    g1_ref[:] = g1t.T


def _stage_b_body(p_ref, wb_ref, g_ref):
    hp = p_ref[0] + p_ref[1]
    g_ref[:] = _kan_packed(hp, [wb_ref[i] for i in range(4)])


def _stage_c_body(p_ref, b_ref, w1c_ref, w1s_ref, w2c_ref, w2s_ref,
                  out_ref):
    hp = p_ref[0] + p_ref[1]
    seg = lax.broadcasted_iota(jnp.int32, (NPAD // 4, NUM_GRAPHS), 1)
    y = jnp.zeros((NUM_GRAPHS, HIDDEN), jnp.float32)
    for j in range(4):
        mj = (b_ref[:, j:j + 1] == seg).astype(jnp.float32)
        y = y + lax.dot_general(
            mj, hp[:, 32 * j:32 * j + 32], (((0,), (0,)), ((), ())),
            preferred_element_type=jnp.float32,
            precision=lax.Precision.HIGHEST)
    yc, ys = _kan_feats(y)
    z = (jnp.dot(yc, w1c_ref[:], preferred_element_type=jnp.float32)
         + jnp.dot(ys, w1s_ref[:], preferred_element_type=jnp.float32))
    z = jnp.where(z >= 0, z, 0.01 * z)
    zc, zs = _kan_feats(z)
    z = (jnp.dot(zc, w2c_ref[:], preferred_element_type=jnp.float32)
         + jnp.dot(zs, w2s_ref[:], preferred_element_type=jnp.float32))
    out_ref[:] = jax.nn.sigmoid(z)


NB = 4
NG = CPW // NB


def _sc_mp_body(g_hbm, src_hbm, dst_hbm, zero_hbm, out_hbm,
                srcs_v, dsts_v, rows_v, stage_v, acc_sh, gsem, ssem):
    c = lax.axis_index("c")
    s = lax.axis_index("s")
    wid = c * NS + s
    pltpu.sync_copy(zero_hbm, stage_v)
    pltpu.sync_copy(stage_v, acc_sh.at[pl.ds(s * RPS, RPS)])
    pltpu.sync_copy(src_hbm.at[:, wid], srcs_v)
    pltpu.sync_copy(dst_hbm.at[:, wid], dsts_v)
    plsc.subcore_barrier()

    def fire_gathers(g, half):
        for b in range(NB):
            pltpu.async_copy(g_hbm.at[srcs_v.at[g * NB + b]],
                             rows_v.at[half].at[b], gsem)

    def drain_gathers(g, half):
        for b in range(NB):
            pltpu.make_async_copy(g_hbm.at[srcs_v.at[g * NB + b]],
                                  rows_v.at[half].at[b], gsem).wait()

    def fire_scatters(g, half):
        for b in range(NB):
            pltpu.async_copy(rows_v.at[half].at[b],
                             acc_sh.at[dsts_v.at[g * NB + b]], ssem, add=True)

    def drain_scatters(g, half):
        for b in range(NB):
            pltpu.make_async_copy(rows_v.at[half].at[b],
                                  acc_sh.at[dsts_v.at[g * NB + b]], ssem).wait()

    fire_gathers(0, 0)

    def body(k, carry):
        g0 = 2 * k

        @pl.when(k > 0)
        def _():
            drain_scatters(g0 - 1, 1)

        fire_gathers(g0 + 1, 1)
        drain_gathers(g0, 0)
        fire_scatters(g0, 0)
        drain_scatters(g0, 0)

        @pl.when(k < NG // 2 - 1)
        def _():
            fire_gathers(g0 + 2, 0)

        drain_gathers(g0 + 1, 1)
        fire_scatters(g0 + 1, 1)
        return carry

    lax.fori_loop(0, NG // 2, body, 0)
    drain_scatters(NG - 1, 1)
    plsc.subcore_barrier()
    pltpu.sync_copy(acc_sh.at[pl.ds(s * RPS, RPS)], stage_v)
    pltpu.sync_copy(stage_v, out_hbm.at[c].at[pl.ds(s * RPS, RPS)])


_sc_mp = functools.partial(
    pl.kernel,
    out_type=jax.ShapeDtypeStruct((NC, NPAD, HIDDEN), jnp.float32),
    mesh=plsc.VectorSubcoreMesh(core_axis_name="c", subcore_axis_name="s"),
    compiler_params=pltpu.CompilerParams(use_tc_tiling_on_sc=False),
    scratch_types=[
        pltpu.VMEM((CPW, CH), jnp.int32),
        pltpu.VMEM((CPW, CH), jnp.int32),
        pltpu.VMEM((2, NB, CH, HIDDEN), jnp.float32),
        pltpu.VMEM((RPS, HIDDEN), jnp.float32),
        pltpu.VMEM_SHARED((NPAD, HIDDEN), jnp.float32),
        pltpu.SemaphoreType.DMA,
        pltpu.SemaphoreType.DMA,
    ],
)(_sc_mp_body)


def kernel(x, edge_index, batch, W_kan, W_l1, W_l2, W1, W2):
    wkc, wks = _kan_pack(W_kan)
    wl1t = jnp.stack([W_l1[h, :, :, g] for h in (0, 1) for g in (0, 1)])
    wl2b = jnp.stack(_kan_pack_blockdiag(W_l2))
    w1c, w1s = _kan_pack(W1)
    w2c, w2s = _kan_pack(W2)
    npk = NPAD // 4

    src = edge_index[0].astype(jnp.int32)
    dst = edge_index[1].astype(jnp.int32)
    n_e = src.shape[0]
    pad = PAD_E - n_e
    src_p = jnp.concatenate(
        [src, jnp.arange(pad, dtype=jnp.int32) % N_NODES]).reshape(CPW, NW, CH)
    dst_p = jnp.concatenate(
        [dst, N_NODES + (jnp.arange(pad, dtype=jnp.int32) % (NPAD - N_NODES))]
    ).reshape(CPW, NW, CH)
    zeros = jnp.zeros((RPS, HIDDEN), jnp.float32)
    batch_p = jnp.concatenate(
        [batch.astype(jnp.int32),
         jnp.full((NPAD - N_NODES,), NUM_GRAPHS, jnp.int32)]
    ).reshape(NPAD // 4, 4)

    blk = 2048
    x_pad = jnp.pad(x, ((0, NPAD - N_NODES), (0, 0)))
    g1p = pl.pallas_call(
        _stage_a_body,
        grid=(NPAD // blk,),
        in_specs=[
            pl.BlockSpec((blk, IN_FEAT), lambda i: (i, 0)),
            pl.BlockSpec((2 * IN_FEAT, HIDDEN), lambda i: (0, 0)),
            pl.BlockSpec((2 * IN_FEAT, HIDDEN), lambda i: (0, 0)),
            pl.BlockSpec((4, HIDDEN, HIDDEN), lambda i: (0, 0, 0)),
        ],
        out_specs=pl.BlockSpec((blk, HIDDEN), lambda i: (i, 0)),
        out_shape=jax.ShapeDtypeStruct((NPAD, HIDDEN), jnp.float32),
    )(x_pad, wkc, wks, wl1t)
    g1 = g1p

    p1 = _sc_mp(g1, src_p, dst_p, zeros)

    g2p = pl.pallas_call(
        _stage_b_body,
        in_specs=[
            pl.BlockSpec((NC, npk, 128), lambda: (0, 0, 0)),
            pl.BlockSpec((4, 128, 128), lambda: (0, 0, 0)),
        ],
        out_specs=pl.BlockSpec((npk, 128), lambda: (0, 0)),
        out_shape=jax.ShapeDtypeStruct((npk, 128), jnp.float32),
    )(p1.reshape(NC, npk, 128), wl2b)
    g2 = g2p.reshape(NPAD, HIDDEN)

    p2 = _sc_mp(g2, src_p, dst_p, zeros)

    out = pl.pallas_call(
        _stage_c_body,
        in_specs=[
            pl.BlockSpec((NC, npk, 128), lambda: (0, 0, 0)),
            pl.BlockSpec((NPAD // 4, 4), lambda: (0, 0)),
            pl.BlockSpec((2 * HIDDEN, HIDDEN), lambda: (0, 0)),
            pl.BlockSpec((2 * HIDDEN, HIDDEN), lambda: (0, 0)),
            pl.BlockSpec((2 * HIDDEN, 1), lambda: (0, 0)),
            pl.BlockSpec((2 * HIDDEN, 1), lambda: (0, 0)),
        ],
        out_specs=pl.BlockSpec((NUM_GRAPHS, 1), lambda: (0, 0)),
        out_shape=jax.ShapeDtypeStruct((NUM_GRAPHS, 1), jnp.float32),
    )(p2.reshape(NC, npk, 128), batch_p, w1c, w1s, w2c, w2s)
    return out
